# baseline (speedup 1.0000x reference)
import functools

import jax
import jax.numpy as jnp
from jax import lax
from jax.experimental import pallas as pl
from jax.experimental.pallas import tpu as pltpu
from jax.experimental.pallas import tpu_sc as plsc

B, N, F, H = 256, 128, 9, 128
NUM_IN_DEG = 512

NW = 32
BATCHES_PER_W = B // NW
NODES_PER_W = BATCHES_PER_W * N
CHUNK = 8
CHUNKS_PER_BATCH = N // CHUNK
NCHUNKS = BATCHES_PER_W * CHUNKS_PER_BATCH
AROWS = CHUNK * F
DROWS = CHUNK * 2
NCOL = H // 16
NBUF = 4


def _body(atom_hbm, degc_hbm, gt_hbm, aidx_hbm, didx_hbm, out_hbm,
          aidx_v, didx_v, a_bufs, d_bufs, o0_v, o1_v, tok_v,
          sa, sd, so0, so1):
    nc = 2
    wid = lax.axis_index("s") * nc + lax.axis_index("c")

    pltpu.sync_copy(aidx_hbm.at[pl.ds(wid * NODES_PER_W * F, NODES_PER_W * F)],
                    aidx_v)
    pltpu.sync_copy(didx_hbm.at[pl.ds(wid * NODES_PER_W * 2, NODES_PER_W * 2)],
                    didx_v)
    pltpu.sync_copy(gt_hbm, tok_v)
    for h in range(NCOL):
        s = pl.ds(h * 16, 16)
        t = tok_v[0, s]
        o0_v[0, s] = t
        o1_v[0, s] = t

    def gather_desc(k, u):
        da = pltpu.make_async_copy(
            atom_hbm.at[aidx_v.at[pl.ds(k * AROWS, AROWS)]], a_bufs[u], sa[u])
        dd = pltpu.make_async_copy(
            degc_hbm.at[didx_v.at[pl.ds(k * DROWS, DROWS)]], d_bufs[u], sd[u])
        return da, dd

    def issue_gather(k, u):
        da, dd = gather_desc(k, u)
        da.start()
        dd.start()

    def wait_gather(k, u):
        da, dd = gather_desc(k, u)
        da.wait()
        dd.wait()

    def compute(c, u, obuf):
        abuf = a_bufs[u]
        dbuf = d_bufs[u]

        def node_body(i, carry):
            for h in range(NCOL):
                s = pl.ds(h * 16, 16)
                t0 = abuf[i * F + 0, s] + abuf[i * F + 1, s]
                t1 = abuf[i * F + 2, s] + abuf[i * F + 3, s]
                t2 = abuf[i * F + 4, s] + abuf[i * F + 5, s]
                t3 = abuf[i * F + 6, s] + abuf[i * F + 7, s]
                t4 = abuf[i * F + 8, s] + dbuf[i * 2, s]
                t5 = dbuf[i * 2 + 1, s]
                obuf[c * CHUNK + i + 1, s] = ((t0 + t1) + (t2 + t3)) + (t4 + t5)
            return carry

        lax.fori_loop(0, CHUNK, node_body, 0)

    def flush_desc(lb, obuf, so):
        return pltpu.make_async_copy(obuf,
                                     out_hbm.at[wid * BATCHES_PER_W + lb], so)

    def process_batch(lb, obuf, so):
        def quad_body(j, carry):
            kq = lb * CHUNKS_PER_BATCH + NBUF * j
            for u in range(NBUF):
                k = kq + u
                kn = jnp.minimum(k + (NBUF - 1), NCHUNKS - 1)
                issue_gather(kn, (u + NBUF - 1) % NBUF)
                wait_gather(k, u)
                compute(NBUF * j + u, u, obuf)
            return carry

        lax.fori_loop(0, CHUNKS_PER_BATCH // NBUF, quad_body, 0)
        flush_desc(lb, obuf, so).start()

    for u in range(NBUF - 1):
        issue_gather(u, u)

    def batch_pair_body(q, carry):
        @pl.when(q > 0)
        def _():
            flush_desc(2 * q, o0_v, so0).wait()

        process_batch(2 * q, o0_v, so0)

        @pl.when(q > 0)
        def _():
            flush_desc(2 * q + 1, o1_v, so1).wait()

        process_batch(2 * q + 1, o1_v, so1)
        return carry

    lax.fori_loop(0, BATCHES_PER_W // 2, batch_pair_body, 0)

    for u in range(NBUF - 1):
        wait_gather(NCHUNKS - 1, u)
    flush_desc(0, o0_v, so0).wait()
    flush_desc(0, o1_v, so1).wait()


@jax.jit
def _run(atom_table, degc, graph_token, aidx, didx):
    mesh = plsc.VectorSubcoreMesh(core_axis_name="c", subcore_axis_name="s")

    def body(atom_hbm, degc_hbm, gt_hbm, aidx_hbm, didx_hbm, out_hbm,
             aidx_v, didx_v, a0, a1, a2, a3, d0, d1, d2, d3, o0_v, o1_v,
             tok_v, sa0, sa1, sa2, sa3, sd0, sd1, sd2, sd3, so0, so1):
        _body(atom_hbm, degc_hbm, gt_hbm, aidx_hbm, didx_hbm, out_hbm,
              aidx_v, didx_v, (a0, a1, a2, a3), (d0, d1, d2, d3),
              o0_v, o1_v, tok_v, (sa0, sa1, sa2, sa3),
              (sd0, sd1, sd2, sd3), so0, so1)

    kfn = functools.partial(
        pl.kernel,
        mesh=mesh,
        out_type=jax.ShapeDtypeStruct((B, N + 1, H), jnp.float32),
        scratch_types=[
            pltpu.VMEM((NODES_PER_W * F,), jnp.int32),
            pltpu.VMEM((NODES_PER_W * 2,), jnp.int32),
        ] + [pltpu.VMEM((AROWS, H), jnp.float32)] * NBUF
        + [pltpu.VMEM((DROWS, H), jnp.float32)] * NBUF
        + [
            pltpu.VMEM((N + 1, H), jnp.float32),
            pltpu.VMEM((N + 1, H), jnp.float32),
            pltpu.VMEM((1, H), jnp.float32),
        ] + [pltpu.SemaphoreType.DMA] * (2 * NBUF + 2),
    )(body)
    return kfn(atom_table, degc, graph_token, aidx, didx)


def kernel(x, in_degree, out_degree, atom_table, in_deg_table, out_deg_table,
           graph_token):
    degc = jnp.concatenate([in_deg_table, out_deg_table], axis=0)
    aidx = x.reshape(-1)
    didx = jnp.stack([in_degree, out_degree + NUM_IN_DEG], axis=-1).reshape(-1)
    return _run(atom_table, degc, graph_token, aidx, didx)
